# baseline (speedup 1.0000x reference)
import functools

import jax
import jax.numpy as jnp
from jax import lax
from jax.experimental import pallas as pl
from jax.experimental.pallas import tpu as pltpu
from jax.experimental.pallas import tpu_sc as plsc

N = 10000
E = 320000
D = 128
NEG = 0.2
DR = 320

NC = 1
NW = 16 * NC
EPW = E // NW
K = 32
NCHUNK = EPW // K
NTRIP = (NCHUNK - 1) // 3
ZR = 80
NBLK = N // ZR

L = 16


def _lin_body(x_ref, wl_ref, wr_ref, xl_ref, xr_ref):
    xb = x_ref[...]
    xl_ref[...] = jnp.dot(xb, wl_ref[...], preferred_element_type=jnp.float32)
    xr_ref[...] = jnp.dot(xb, wr_ref[...], preferred_element_type=jnp.float32)


def _node_transform(x, W_l, W_r):
    blk = 1000
    return pl.pallas_call(
        _lin_body,
        grid=(N // blk,),
        in_specs=[
            pl.BlockSpec((blk, D), lambda i: (i, 0)),
            pl.BlockSpec((D, D), lambda i: (0, 0)),
            pl.BlockSpec((D, D), lambda i: (0, 0)),
        ],
        out_specs=[
            pl.BlockSpec((blk, D), lambda i: (i, 0)),
            pl.BlockSpec((blk, D), lambda i: (i, 0)),
        ],
        out_shape=[
            jax.ShapeDtypeStruct((N, D), jnp.float32),
            jax.ShapeDtypeStruct((N, D), jnp.float32),
        ],
    )(x, W_l, W_r)


def _edge_body(ea_ref, we_ref, ef_ref):
    ef_ref[...] = jnp.dot(ea_ref[...], we_ref[...],
                          preferred_element_type=jnp.float32)


def _edge_transform(edge_attr, W_e):
    blk = 4000
    de = edge_attr.shape[1]
    return pl.pallas_call(
        _edge_body,
        grid=(E // blk,),
        in_specs=[
            pl.BlockSpec((blk, de), lambda i: (i, 0)),
            pl.BlockSpec((de, D), lambda i: (0, 0)),
        ],
        out_specs=pl.BlockSpec((blk, D), lambda i: (i, 0)),
        out_shape=jax.ShapeDtypeStruct((E, D), jnp.float32),
    )(edge_attr, W_e)


def _lane_gather(v, idx):
    dnums = lax.GatherDimensionNumbers(
        offset_dims=(), collapsed_slice_dims=(0,), start_index_map=(0,))
    return lax.gather(v, idx[:, None], dnums, slice_sizes=(1,),
                      mode=lax.GatherScatterMode.PROMISE_IN_BOUNDS)


def _sc_edge_phase(xl, xr, ef, src, dst, att_flat):
    mesh = plsc.VectorSubcoreMesh(core_axis_name="c", subcore_axis_name="s",
                                  num_cores=NC)

    @functools.partial(
        pl.kernel,
        mesh=mesh,
        out_type=[
            jax.ShapeDtypeStruct((NC * N, D), jnp.float32),
            jax.ShapeDtypeStruct((NC, DR, D), jnp.float32),
        ],
        scratch_types=(
            [pltpu.VMEM((K,), jnp.int32)] * 12 +
            [pltpu.VMEM((K, D), jnp.float32)] * 9 +
            [pltpu.VMEM((D,), jnp.float32)] +
            [pltpu.VMEM_SHARED((N, D), jnp.float32),
             pltpu.VMEM_SHARED((DR, D), jnp.float32)]
            + [pltpu.SemaphoreType.DMA] * 12
        ),
    )
    def sc_kernel(xl_hbm, xr_hbm, ef_hbm, src_hbm, dst_hbm, att_hbm,
                  num_hbm, den_hbm,
                  src0, src1, src2, dst0, dst1, dst2,
                  dsc0, dsc1, dsc2, drow0, drow1, drow2,
                  xl0, xl1, xl2, xr0, xr1, xr2, ef0, ef1, ef2,
                  att_v, acc_sh, den_sh,
                  gsem0, gsem1, gsem2, isem0, isem1, isem2,
                  nsem0, nsem1, nsem2, dsem0, dsem1, dsem2):
        cid = lax.axis_index("c")
        sid = lax.axis_index("s")
        wid = sid * NC + cid
        base0 = wid * EPW

        SRC = (src0, src1, src2)
        DST = (dst0, dst1, dst2)
        DSC = (dsc0, dsc1, dsc2)
        DROW = (drow0, drow1, drow2)
        XL = (xl0, xl1, xl2)
        XR = (xr0, xr1, xr2)
        EF = (ef0, ef1, ef2)
        GSEM = (gsem0, gsem1, gsem2)
        ISEM = (isem0, isem1, isem2)
        NSEM = (nsem0, nsem1, nsem2)
        DSEM = (dsem0, dsem1, dsem2)

        def fire_idx(ci, P):
            base = base0 + ci * K
            pltpu.async_copy(src_hbm.at[pl.ds(base, K)], SRC[P], ISEM[P])
            pltpu.async_copy(dst_hbm.at[pl.ds(base, K)], DST[P], ISEM[P])

        def wait_idx(P):
            sem = ISEM[P]
            pltpu.make_async_copy(src_hbm.at[pl.ds(0, K)], SRC[P], sem).wait()
            pltpu.make_async_copy(src_hbm.at[pl.ds(0, K)], DST[P], sem).wait()

        def fire_rows(ci, P):
            pltpu.async_copy(xl_hbm.at[SRC[P]], XL[P], GSEM[P])
            pltpu.async_copy(xr_hbm.at[DST[P]], XR[P], GSEM[P])
            pltpu.async_copy(ef_hbm.at[pl.ds(base0 + ci * K, K)],
                             EF[P], GSEM[P])

        def wait_rows(P):
            pltpu.make_async_copy(xl_hbm.at[pl.ds(0, K)], XL[P],
                                  GSEM[P]).wait()
            pltpu.make_async_copy(xl_hbm.at[pl.ds(0, K)], XR[P],
                                  GSEM[P]).wait()
            pltpu.make_async_copy(ef_hbm.at[pl.ds(0, K)], EF[P],
                                  GSEM[P]).wait()

        def wait_num(P):
            pltpu.make_async_copy(xl_hbm.at[pl.ds(0, K)], XL[P],
                                  NSEM[P]).wait()

        def wait_den(P):
            pltpu.make_async_copy(xl_hbm.at[pl.ds(0, K)], XR[P],
                                  DSEM[P]).wait()

        fire_idx(0, 0)
        fire_idx(1, 1)
        fire_idx(2, 2)

        zvec = jnp.zeros((L,), jnp.float32)

        def zrow(r, carry):
            for j in range(D // L):
                xr0[r, pl.ds(j * L, L)] = zvec
            return carry

        lax.fori_loop(0, K, zrow, 0)

        for i in range((N // K + 15) // 16):
            b = sid + 16 * i
            @pl.when(b < N // K)
            def _():
                pltpu.sync_copy(xr0, acc_sh.at[pl.ds(b * K, K)])

        @pl.when(sid == 0)
        def _():
            pltpu.sync_copy(xr0.at[pl.ds(0, N - (N // K) * K)],
                            acc_sh.at[pl.ds((N // K) * K, N - (N // K) * K)])

        @pl.when(sid == 1)
        def _():
            for i in range(DR // K):
                pltpu.sync_copy(xr0, den_sh.at[pl.ds(i * K, K)])

        pltpu.sync_copy(att_hbm, att_v)

        wait_idx(0)
        fire_rows(0, 0)
        wait_idx(1)
        fire_rows(1, 1)

        plsc.subcore_barrier()

        att_js = [att_v[pl.ds(j * L, L)] for j in range(D // L)]
        lanes = lax.iota(jnp.int32, L)
        shuf = [(lanes + s) & (L - 1) for s in (8, 4, 2, 1)]

        def compute(P):
            xl_v, xr_v, ef_v = XL[P], XR[P], EF[P]
            dst_v, dsc_v, drow_v = DST[P], DSC[P], DROW[P]

            def group_body(g, gcarry):
                dstg = dst_v[pl.ds(g * L, L)]
                dsc_v[pl.ds(g * L, L)] = dstg
                drow_v[pl.ds(g * L, L)] = lax.shift_right_logical(dstg, 5)
                colg = lax.bitwise_and(dstg, jnp.int32(2 * L - 1))
                lg = jnp.zeros((L,), jnp.float32)
                for i in range(L):
                    k = g * L + i
                    ps = []
                    for j in range(D // L):
                        t = (xl_v[k, pl.ds(j * L, L)]
                             + xr_v[k, pl.ds(j * L, L)]
                             + ef_v[k, pl.ds(j * L, L)])
                        t = jnp.maximum(t, t * NEG)
                        ps.append(t * att_js[j])
                    while len(ps) > 1:
                        ps = [ps[m] + ps[m + 1] for m in range(0, len(ps), 2)]
                    acc = ps[0]
                    for sidx in shuf:
                        acc = acc + _lane_gather(acc, sidx)
                    lg = jnp.where(lanes == i, acc, lg)
                wg = jnp.exp(lg)
                for i in range(L):
                    k = g * L + i
                    wv = _lane_gather(wg, jnp.full((L,), i, jnp.int32))
                    for j in range(D // L):
                        xl_v[k, pl.ds(j * L, L)] = (
                            wv * xl_v[k, pl.ds(j * L, L)])
                    colb = _lane_gather(colg, jnp.full((L,), i, jnp.int32))
                    xr_v[k, pl.ds(0, L)] = jnp.where(lanes == colb, wv, 0.0)
                    xr_v[k, pl.ds(L, L)] = jnp.where(
                        lanes + L == colb, wv, 0.0)
                return gcarry

            lax.fori_loop(0, K // L, group_body, 0)

        def fire_scatters(P):
            pltpu.async_copy(XL[P], acc_sh.at[DSC[P]], NSEM[P], add=True)
            pltpu.async_copy(XR[P], den_sh.at[DROW[P]], DSEM[P], add=True)

        def slot(ci, X, st4_guard, adv_guard, idx_guard):
            X2 = (X + 2) % 3
            wait_rows(X)
            compute(X)
            fire_scatters(X)

            def scat_waits():
                wait_num(X2)
                wait_den(X2)

            def advance():
                wait_idx(X2)
                fire_rows(ci + 2, X2)

            if st4_guard is None:
                scat_waits()
            else:
                @pl.when(st4_guard)
                def _():
                    scat_waits()
            if adv_guard is None:
                advance()
            else:
                @pl.when(adv_guard)
                def _():
                    advance()
            if idx_guard is None:
                fire_idx(ci + 3, X)
            else:
                @pl.when(idx_guard)
                def _():
                    fire_idx(ci + 3, X)

        def triple_body(p, carry):
            c0 = 3 * p
            last = p < NTRIP - 1
            slot(c0, 0, p > 0, None, None)
            slot(c0 + 1, 1, None, None, last)
            slot(c0 + 2, 2, last, last, last)
            return carry

        lax.fori_loop(0, NTRIP, triple_body, 0)

        wait_rows(0)
        compute(0)
        fire_scatters(0)

        wait_num(1)
        wait_den(1)
        wait_num(2)
        wait_den(2)
        wait_num(0)
        wait_den(0)

        plsc.subcore_barrier()
        for i in range((NBLK + 15) // 16):
            b = sid + 16 * i
            @pl.when(b < NBLK)
            def _():
                pltpu.sync_copy(acc_sh.at[pl.ds(b * ZR, ZR)],
                                num_hbm.at[pl.ds(cid * N + b * ZR, ZR)])

        @pl.when(sid == 0)
        def _():
            pltpu.sync_copy(den_sh, den_hbm.at[cid])

    return sc_kernel(xl, xr, ef, src, dst, att_flat)


def _final_body(num_ref, den_ref, bias_ref, out_ref):
    num = num_ref[0]
    den = den_ref[0]
    for c in range(1, NC):
        num = num + num_ref[c]
        den = den + den_ref[c]
    out_ref[...] = num / (den + 1e-16) + bias_ref[...]


def _finalize(num2, den2, bias2):
    blk = 1000
    return pl.pallas_call(
        _final_body,
        grid=(N // blk,),
        in_specs=[
            pl.BlockSpec((NC, blk, D), lambda i: (0, i, 0)),
            pl.BlockSpec((NC, blk, 1), lambda i: (0, i, 0)),
            pl.BlockSpec((1, D), lambda i: (0, 0)),
        ],
        out_specs=pl.BlockSpec((blk, D), lambda i: (i, 0)),
        out_shape=jax.ShapeDtypeStruct((N, D), jnp.float32),
    )(num2, den2, bias2)


def kernel(x, edge_index, edge_attr, W_l, W_r, W_e, att, bias):
    src = edge_index[0].astype(jnp.int32)
    dst = edge_index[1].astype(jnp.int32)
    xl, xr = _node_transform(x, W_l, W_r)
    ef = _edge_transform(edge_attr, W_e)
    num, den = _sc_edge_phase(xl, xr, ef, src, dst, att.reshape(D))
    den2 = den[:, :, :2 * 16].reshape(NC, DR * 2 * 16)[:, :N].reshape(
        NC, N, 1)
    out = _finalize(num.reshape(NC, N, D), den2, bias.reshape(1, D))
    return out


# speedup vs baseline: 1.2948x; 1.2948x over previous
import functools

import jax
import jax.numpy as jnp
from jax import lax
from jax.experimental import pallas as pl
from jax.experimental.pallas import tpu as pltpu
from jax.experimental.pallas import tpu_sc as plsc

N = 10000
E = 320000
D = 128
NEG = 0.2
DR = 320

NC = 1
NW = 16 * NC
EPW = E // NW
K = 32
NCHUNK = EPW // K
NTRIP = (NCHUNK - 1) // 3
ZR = 80
NBLK = N // ZR

L = 16


def _lin_body(x_ref, wl_ref, wr_ref, xl_ref, xr_ref):
    xb = x_ref[...]
    xl_ref[...] = jnp.dot(xb, wl_ref[...], preferred_element_type=jnp.float32)
    xr_ref[...] = jnp.dot(xb, wr_ref[...], preferred_element_type=jnp.float32)


def _node_transform(x, W_l, W_r):
    blk = 1000
    return pl.pallas_call(
        _lin_body,
        grid=(N // blk,),
        in_specs=[
            pl.BlockSpec((blk, D), lambda i: (i, 0)),
            pl.BlockSpec((D, D), lambda i: (0, 0)),
            pl.BlockSpec((D, D), lambda i: (0, 0)),
        ],
        out_specs=[
            pl.BlockSpec((blk, D), lambda i: (i, 0)),
            pl.BlockSpec((blk, D), lambda i: (i, 0)),
        ],
        out_shape=[
            jax.ShapeDtypeStruct((N, D), jnp.float32),
            jax.ShapeDtypeStruct((N, D), jnp.float32),
        ],
    )(x, W_l, W_r)


def _edge_body(ea_ref, we_ref, ef_ref):
    ef_ref[...] = jnp.dot(ea_ref[...], we_ref[...],
                          preferred_element_type=jnp.float32)


def _edge_transform(edge_attr, W_e):
    blk = 4000
    de = edge_attr.shape[1]
    return pl.pallas_call(
        _edge_body,
        grid=(E // blk,),
        in_specs=[
            pl.BlockSpec((blk, de), lambda i: (i, 0)),
            pl.BlockSpec((de, D), lambda i: (0, 0)),
        ],
        out_specs=pl.BlockSpec((blk, D), lambda i: (i, 0)),
        out_shape=jax.ShapeDtypeStruct((E, D), jnp.float32),
    )(edge_attr, W_e)


def _lane_gather(v, idx):
    dnums = lax.GatherDimensionNumbers(
        offset_dims=(), collapsed_slice_dims=(0,), start_index_map=(0,))
    return lax.gather(v, idx[:, None], dnums, slice_sizes=(1,),
                      mode=lax.GatherScatterMode.PROMISE_IN_BOUNDS)


def _sc_edge_phase(xl, xr, ef, src, dst, att_flat):
    mesh = plsc.VectorSubcoreMesh(core_axis_name="c", subcore_axis_name="s",
                                  num_cores=NC)

    @functools.partial(
        pl.kernel,
        mesh=mesh,
        out_type=[
            jax.ShapeDtypeStruct((NC * N, D), jnp.float32),
            jax.ShapeDtypeStruct((NC, DR, D), jnp.float32),
        ],
        scratch_types=(
            [pltpu.VMEM((K,), jnp.int32)] * 12 +
            [pltpu.VMEM((K, D), jnp.float32)] * 9 +
            [pltpu.VMEM((D,), jnp.float32)] +
            [pltpu.VMEM_SHARED((N, D), jnp.float32),
             pltpu.VMEM_SHARED((DR, D), jnp.float32)]
            + [pltpu.SemaphoreType.DMA] * 12
        ),
    )
    def sc_kernel(xl_hbm, xr_hbm, ef_hbm, src_hbm, dst_hbm, att_hbm,
                  num_hbm, den_hbm,
                  src0, src1, src2, dst0, dst1, dst2,
                  dsc0, dsc1, dsc2, drow0, drow1, drow2,
                  xl0, xl1, xl2, xr0, xr1, xr2, ef0, ef1, ef2,
                  att_v, acc_sh, den_sh,
                  gsem0, gsem1, gsem2, isem0, isem1, isem2,
                  nsem0, nsem1, nsem2, dsem0, dsem1, dsem2):
        cid = lax.axis_index("c")
        sid = lax.axis_index("s")
        wid = sid * NC + cid
        base0 = wid * EPW

        SRC = (src0, src1, src2)
        DST = (dst0, dst1, dst2)
        DSC = (dsc0, dsc1, dsc2)
        DROW = (drow0, drow1, drow2)
        XL = (xl0, xl1, xl2)
        XR = (xr0, xr1, xr2)
        EF = (ef0, ef1, ef2)
        GSEM = (gsem0, gsem1, gsem2)
        ISEM = (isem0, isem1, isem2)
        NSEM = (nsem0, nsem1, nsem2)
        DSEM = (dsem0, dsem1, dsem2)

        def fire_idx(ci, P):
            base = base0 + ci * K
            pltpu.async_copy(src_hbm.at[pl.ds(base, K)], SRC[P], ISEM[P])
            pltpu.async_copy(dst_hbm.at[pl.ds(base, K)], DST[P], ISEM[P])

        def wait_idx(P):
            sem = ISEM[P]
            pltpu.make_async_copy(src_hbm.at[pl.ds(0, K)], SRC[P], sem).wait()
            pltpu.make_async_copy(src_hbm.at[pl.ds(0, K)], DST[P], sem).wait()

        def fire_rows(ci, P):
            pltpu.async_copy(xl_hbm.at[SRC[P]], XL[P], GSEM[P])
            pltpu.async_copy(xr_hbm.at[DST[P]], XR[P], GSEM[P])
            pltpu.async_copy(ef_hbm.at[pl.ds(base0 + ci * K, K)],
                             EF[P], GSEM[P])

        def wait_rows(P):
            pltpu.make_async_copy(xl_hbm.at[pl.ds(0, K)], XL[P],
                                  GSEM[P]).wait()
            pltpu.make_async_copy(xl_hbm.at[pl.ds(0, K)], XR[P],
                                  GSEM[P]).wait()
            pltpu.make_async_copy(ef_hbm.at[pl.ds(0, K)], EF[P],
                                  GSEM[P]).wait()

        def wait_num(P):
            pltpu.make_async_copy(xl_hbm.at[pl.ds(0, K)], XL[P],
                                  NSEM[P]).wait()

        def wait_den(P):
            pltpu.make_async_copy(xl_hbm.at[pl.ds(0, K)], XR[P],
                                  DSEM[P]).wait()

        fire_idx(0, 0)
        fire_idx(1, 1)
        fire_idx(2, 2)

        zvec = jnp.zeros((L,), jnp.float32)

        def zrow(r, carry):
            for j in range(D // L):
                xr0[r, pl.ds(j * L, L)] = zvec
            return carry

        lax.fori_loop(0, K, zrow, 0)

        for i in range((N // K + 15) // 16):
            b = sid + 16 * i
            @pl.when(b < N // K)
            def _():
                pltpu.sync_copy(xr0, acc_sh.at[pl.ds(b * K, K)])

        @pl.when(sid == 0)
        def _():
            pltpu.sync_copy(xr0.at[pl.ds(0, N - (N // K) * K)],
                            acc_sh.at[pl.ds((N // K) * K, N - (N // K) * K)])

        @pl.when(sid == 1)
        def _():
            for i in range(DR // K):
                pltpu.sync_copy(xr0, den_sh.at[pl.ds(i * K, K)])

        pltpu.sync_copy(att_hbm, att_v)

        wait_idx(0)
        fire_rows(0, 0)
        wait_idx(1)
        fire_rows(1, 1)

        plsc.subcore_barrier()

        att_js = [att_v[pl.ds(j * L, L)] for j in range(D // L)]
        lanes = lax.iota(jnp.int32, L)
        shuf = [(lanes + s) & (L - 1) for s in (8, 4, 2, 1)]

        def compute(P):
            xl_v, xr_v, ef_v = XL[P], XR[P], EF[P]
            dst_v, dsc_v, drow_v = DST[P], DSC[P], DROW[P]

            def group_body(g, gcarry):
                dstg = dst_v[pl.ds(g * L, L)]
                dsc_v[pl.ds(g * L, L)] = dstg
                drow_v[pl.ds(g * L, L)] = lax.shift_right_logical(dstg, 5)
                colg = lax.bitwise_and(dstg, jnp.int32(2 * L - 1))
                for i in range(L):
                    k = g * L + i
                    xs = []
                    acc = jnp.zeros((L,), jnp.float32)
                    for j in range(D // L):
                        a = xl_v[k, pl.ds(j * L, L)]
                        xs.append(a)
                        t = (a + xr_v[k, pl.ds(j * L, L)]
                             + ef_v[k, pl.ds(j * L, L)])
                        t = jnp.maximum(t, t * NEG)
                        acc = acc + t * att_js[j]
                    for sidx in shuf:
                        acc = acc + _lane_gather(acc, sidx)
                    wv = jnp.exp(acc)
                    for j in range(D // L):
                        xl_v[k, pl.ds(j * L, L)] = wv * xs[j]
                    colb = _lane_gather(colg, jnp.full((L,), i, jnp.int32))
                    xr_v[k, pl.ds(0, L)] = jnp.where(lanes == colb, wv, 0.0)
                    xr_v[k, pl.ds(L, L)] = jnp.where(
                        lanes + L == colb, wv, 0.0)
                return gcarry

            lax.fori_loop(0, K // L, group_body, 0)

        def fire_scatters(P):
            pltpu.async_copy(XL[P], acc_sh.at[DSC[P]], NSEM[P], add=True)
            pltpu.async_copy(XR[P], den_sh.at[DROW[P]], DSEM[P], add=True)

        def slot(ci, X, st4_guard, adv_guard, idx_guard):
            X2 = (X + 2) % 3
            wait_rows(X)
            compute(X)
            fire_scatters(X)

            def scat_waits():
                wait_num(X2)
                wait_den(X2)

            def advance():
                wait_idx(X2)
                fire_rows(ci + 2, X2)

            if st4_guard is None:
                scat_waits()
            else:
                @pl.when(st4_guard)
                def _():
                    scat_waits()
            if adv_guard is None:
                advance()
            else:
                @pl.when(adv_guard)
                def _():
                    advance()
            if idx_guard is None:
                fire_idx(ci + 3, X)
            else:
                @pl.when(idx_guard)
                def _():
                    fire_idx(ci + 3, X)

        def triple_body(p, carry):
            c0 = 3 * p
            last = p < NTRIP - 1
            slot(c0, 0, p > 0, None, None)
            slot(c0 + 1, 1, None, None, last)
            slot(c0 + 2, 2, last, last, last)
            return carry

        lax.fori_loop(0, NTRIP, triple_body, 0)

        wait_rows(0)
        compute(0)
        fire_scatters(0)

        wait_num(1)
        wait_den(1)
        wait_num(2)
        wait_den(2)
        wait_num(0)
        wait_den(0)

        plsc.subcore_barrier()
        for i in range((NBLK + 15) // 16):
            b = sid + 16 * i
            @pl.when(b < NBLK)
            def _():
                pltpu.sync_copy(acc_sh.at[pl.ds(b * ZR, ZR)],
                                num_hbm.at[pl.ds(cid * N + b * ZR, ZR)])

        @pl.when(sid == 0)
        def _():
            pltpu.sync_copy(den_sh, den_hbm.at[cid])

    return sc_kernel(xl, xr, ef, src, dst, att_flat)


def _final_body(num_ref, den_ref, bias_ref, out_ref):
    num = num_ref[0]
    den = den_ref[0]
    for c in range(1, NC):
        num = num + num_ref[c]
        den = den + den_ref[c]
    out_ref[...] = num / (den + 1e-16) + bias_ref[...]


def _finalize(num2, den2, bias2):
    blk = 1000
    return pl.pallas_call(
        _final_body,
        grid=(N // blk,),
        in_specs=[
            pl.BlockSpec((NC, blk, D), lambda i: (0, i, 0)),
            pl.BlockSpec((NC, blk, 1), lambda i: (0, i, 0)),
            pl.BlockSpec((1, D), lambda i: (0, 0)),
        ],
        out_specs=pl.BlockSpec((blk, D), lambda i: (i, 0)),
        out_shape=jax.ShapeDtypeStruct((N, D), jnp.float32),
    )(num2, den2, bias2)


def kernel(x, edge_index, edge_attr, W_l, W_r, W_e, att, bias):
    src = edge_index[0].astype(jnp.int32)
    dst = edge_index[1].astype(jnp.int32)
    xl, xr = _node_transform(x, W_l, W_r)
    ef = _edge_transform(edge_attr, W_e)
    num, den = _sc_edge_phase(xl, xr, ef, src, dst, att.reshape(D))
    den2 = den[:, :, :2 * 16].reshape(NC, DR * 2 * 16)[:, :N].reshape(
        NC, N, 1)
    out = _finalize(num.reshape(NC, N, D), den2, bias.reshape(1, D))
    return out
